# baseline (speedup 1.0000x reference)
import functools

import jax
import jax.numpy as jnp
from jax import lax
from jax.experimental import pallas as pl
from jax.experimental.pallas import tpu as pltpu, tpu_sc as plsc

B, C, H, W = 64, 384, 28, 28
NC, NS, L = 2, 16, 16
NW = NC * NS
BPW = B // NW
K = 12
NCHUNK = C // K
V16 = K // L


def _body(x_hbm, perm_hbm, out_hbm,
          perm_v, buf0, buf1, gs0, gs1, ws0, ws1):
    wid = lax.axis_index("s") * NC + lax.axis_index("c")

    pltpu.sync_copy(perm_hbm, perm_v)

    bufs = (buf0, buf1)
    gsems = (gs0, gs1)
    wsems = (ws0, ws1)

    def copy_in(b, i, s):
        handles = []
        o = min(i * K, C - L)
        vv = perm_v[pl.ds(o, L)]
        for j in range(K):
            p = vv[i * K - o + j]
            handles.append(pltpu.async_copy(
                x_hbm.at[b].at[pl.ds(p, 1)],
                bufs[s].at[pl.ds(j, 1)], gsems[s]))
        return handles

    def copy_out(b, i, s):
        return pltpu.async_copy(bufs[s], out_hbm.at[b, pl.ds(i * K, K)],
                                wsems[s])

    for bl in range(BPW):
        b = wid * BPW + bl
        g = {}
        w = {}
        g[0] = copy_in(b, 0, 0)
        for i in range(NCHUNK):
            s = i % 2
            if i + 1 < NCHUNK:
                if i - 1 >= 0:
                    w[i - 1].wait()
                g[i + 1] = copy_in(b, i + 1, (i + 1) % 2)
            for h in g.pop(i):
                h.wait()
            w[i] = copy_out(b, i, s)
        w[NCHUNK - 2].wait()
        w[NCHUNK - 1].wait()


@jax.jit
def _permute(x, perm):
    mesh = plsc.VectorSubcoreMesh(core_axis_name="c", subcore_axis_name="s")
    run = functools.partial(
        pl.kernel,
        mesh=mesh,
        out_type=jax.ShapeDtypeStruct((B, C, H, W), jnp.float32),
        scratch_types=[
            pltpu.VMEM((C,), jnp.int32),
            pltpu.VMEM((K, H, W), jnp.float32),
            pltpu.VMEM((K, H, W), jnp.float32),
            pltpu.SemaphoreType.DMA,
            pltpu.SemaphoreType.DMA,
            pltpu.SemaphoreType.DMA,
            pltpu.SemaphoreType.DMA,
        ],
    )(_body)
    return run(x, perm)


def kernel(x, perm):
    y = _permute(x, perm)
    logdet = jnp.zeros((B,), dtype=x.dtype)
    return (y, logdet)


# speedup vs baseline: 1.0149x; 1.0149x over previous
import jax
import jax.numpy as jnp
from jax import lax
from jax.experimental import pallas as pl
from jax.experimental.pallas import tpu as pltpu

B, C, H, W = 64, 384, 28, 28


def _tc_body(perm_ref, x_ref, o_ref):
    def step(c, _):
        p = perm_ref[c]
        o_ref[0, pl.ds(c, 1)] = x_ref[0, pl.ds(p, 1)]
        return 0

    lax.fori_loop(0, C, step, 0)


@jax.jit
def _permute(x, perm):
    grid_spec = pltpu.PrefetchScalarGridSpec(
        num_scalar_prefetch=1,
        grid=(B,),
        in_specs=[
            pl.BlockSpec((1, C, H, W), lambda i, perm_ref: (i, 0, 0, 0)),
        ],
        out_specs=pl.BlockSpec((1, C, H, W), lambda i, perm_ref: (i, 0, 0, 0)),
    )
    return pl.pallas_call(
        _tc_body,
        grid_spec=grid_spec,
        out_shape=jax.ShapeDtypeStruct((B, C, H, W), jnp.float32),
    )(perm, x)


def kernel(x, perm):
    y = _permute(x, perm)
    logdet = jnp.zeros((B,), dtype=x.dtype)
    return (y, logdet)


# speedup vs baseline: 1.0413x; 1.0260x over previous
import functools

import jax
import jax.numpy as jnp
from jax import lax
from jax.experimental import pallas as pl
from jax.experimental.pallas import tpu as pltpu, tpu_sc as plsc

B, C, H, W = 64, 384, 28, 28
NC, NS, L = 2, 16, 16
NW = NC * NS
BT = 32
BS = B - BT
UNROLL = 4
K = 64
NCHUNK = C // K


def _tc_body(perm_ref, x_ref, o_ref):
    def step(c0, _):
        for u in range(UNROLL):
            c = c0 * UNROLL + u
            p = perm_ref[c]
            o_ref[0, pl.ds(c, 1)] = x_ref[0, pl.ds(p, 1)]
        return 0

    lax.fori_loop(0, C // UNROLL, step, 0)


def _tc_permute(x_t, perm):
    grid_spec = pltpu.PrefetchScalarGridSpec(
        num_scalar_prefetch=1,
        grid=(BT,),
        in_specs=[
            pl.BlockSpec((1, C, H, W), lambda i, perm_ref: (i, 0, 0, 0)),
        ],
        out_specs=pl.BlockSpec((1, C, H, W), lambda i, perm_ref: (i, 0, 0, 0)),
    )
    return pl.pallas_call(
        _tc_body,
        grid_spec=grid_spec,
        out_shape=jax.ShapeDtypeStruct((BT, C, H, W), jnp.float32),
    )(perm, x_t)


def _sc_body(x_hbm, perm_hbm, out_hbm,
             perm_v, buf0, buf1, gs0, gs1, ws0, ws1):
    wid = lax.axis_index("s") * NC + lax.axis_index("c")

    pltpu.sync_copy(perm_hbm, perm_v)

    bufs = (buf0, buf1)
    gsems = (gs0, gs1)
    wsems = (ws0, ws1)

    def copy_in(b, i, s):
        return pltpu.async_copy(
            x_hbm.at[b].at[perm_v.at[pl.ds(i * K, K)]], bufs[s], gsems[s])

    def copy_out(b, i, s):
        return pltpu.async_copy(bufs[s], out_hbm.at[b, pl.ds(i * K, K)],
                                wsems[s])

    b = wid
    g = {}
    w = {}
    g[0] = copy_in(b, 0, 0)
    for i in range(NCHUNK):
        s = i % 2
        if i + 1 < NCHUNK:
            if i - 1 >= 0:
                w[i - 1].wait()
            g[i + 1] = copy_in(b, i + 1, (i + 1) % 2)
        g[i].wait()
        w[i] = copy_out(b, i, s)
    w[NCHUNK - 2].wait()
    w[NCHUNK - 1].wait()


def _sc_permute(x_s, perm):
    mesh = plsc.VectorSubcoreMesh(core_axis_name="c", subcore_axis_name="s")
    run = functools.partial(
        pl.kernel,
        mesh=mesh,
        compiler_params=pltpu.CompilerParams(use_tc_tiling_on_sc=False),
        out_type=jax.ShapeDtypeStruct((BS, C, H * W), jnp.float32),
        scratch_types=[
            pltpu.VMEM((C,), jnp.int32),
            pltpu.VMEM((K, H * W), jnp.float32),
            pltpu.VMEM((K, H * W), jnp.float32),
            pltpu.SemaphoreType.DMA,
            pltpu.SemaphoreType.DMA,
            pltpu.SemaphoreType.DMA,
            pltpu.SemaphoreType.DMA,
        ],
    )(_sc_body)
    return run(x_s.reshape(BS, C, H * W), perm)


@jax.jit
def _permute(x, perm):
    y_sc = _sc_permute(x[BT:], perm).reshape(BS, C, H, W)
    y_tc = _tc_permute(x[:BT], perm)
    return jnp.concatenate([y_tc, y_sc], axis=0)


def kernel(x, perm):
    y = _permute(x, perm)
    logdet = jnp.zeros((B,), dtype=x.dtype)
    return (y, logdet)


# speedup vs baseline: 1.6004x; 1.5370x over previous
import functools

import jax
import jax.numpy as jnp
from jax import lax
from jax.experimental import pallas as pl
from jax.experimental.pallas import tpu as pltpu, tpu_sc as plsc

B, C, H, W = 64, 384, 28, 28
NC, NS, L = 2, 16, 16
NW = NC * NS
BPW = B // NW
K = 64
NCHUNK = C // K


def _body(x_hbm, perm_hbm, out_hbm,
          perm_v, buf0, buf1, gs0, gs1, ws0, ws1):
    wid = lax.axis_index("s") * NC + lax.axis_index("c")

    pltpu.sync_copy(perm_hbm, perm_v)

    bufs = (buf0, buf1)
    gsems = (gs0, gs1)
    wsems = (ws0, ws1)

    def copy_in(b, i, s):
        return pltpu.async_copy(
            x_hbm.at[b].at[perm_v.at[pl.ds(i * K, K)]], bufs[s], gsems[s])

    def copy_out(b, i, s):
        return pltpu.async_copy(bufs[s], out_hbm.at[b, pl.ds(i * K, K)],
                                wsems[s])

    total = BPW * NCHUNK
    chunks = [(wid * BPW + j // NCHUNK, j % NCHUNK) for j in range(total)]
    g = {}
    w = {}
    g[0] = copy_in(*chunks[0], 0)
    for j in range(total):
        s = j % 2
        if j + 1 < total:
            if j - 1 >= 0:
                w[j - 1].wait()
            g[j + 1] = copy_in(*chunks[j + 1], (j + 1) % 2)
        g[j].wait()
        w[j] = copy_out(*chunks[j], s)
    w[total - 2].wait()
    w[total - 1].wait()


@jax.jit
def _permute(x, perm):
    mesh = plsc.VectorSubcoreMesh(core_axis_name="c", subcore_axis_name="s")
    run = functools.partial(
        pl.kernel,
        mesh=mesh,
        compiler_params=pltpu.CompilerParams(use_tc_tiling_on_sc=False),
        out_type=jax.ShapeDtypeStruct((B, C, H * W), jnp.float32),
        scratch_types=[
            pltpu.VMEM((C,), jnp.int32),
            pltpu.VMEM((K, H * W), jnp.float32),
            pltpu.VMEM((K, H * W), jnp.float32),
            pltpu.SemaphoreType.DMA,
            pltpu.SemaphoreType.DMA,
            pltpu.SemaphoreType.DMA,
            pltpu.SemaphoreType.DMA,
        ],
    )(_body)
    return run(x.reshape(B, C, H * W), perm)


def kernel(x, perm):
    y = _permute(x, perm).reshape(B, C, H, W)
    logdet = jnp.zeros((B,), dtype=x.dtype)
    return (y, logdet)


# speedup vs baseline: 1.6023x; 1.0012x over previous
import functools

import jax
import jax.numpy as jnp
from jax import lax
from jax.experimental import pallas as pl
from jax.experimental.pallas import tpu as pltpu, tpu_sc as plsc

B, C, H, W = 64, 384, 28, 28
NC, NS, L = 2, 16, 16
NW = NC * NS
BPW = B // NW
K = 32
NCHUNK = C // K


def _body(x_hbm, perm_hbm, out_hbm,
          perm_v, buf0, buf1, buf2, gs0, gs1, gs2, ws0, ws1, ws2):
    wid = lax.axis_index("s") * NC + lax.axis_index("c")

    pltpu.sync_copy(perm_hbm, perm_v)

    bufs = (buf0, buf1, buf2)
    gsems = (gs0, gs1, gs2)
    wsems = (ws0, ws1, ws2)

    def copy_in(b, i, s):
        return pltpu.async_copy(
            x_hbm.at[b].at[perm_v.at[pl.ds(i * K, K)]], bufs[s], gsems[s])

    def copy_out(b, i, s):
        return pltpu.async_copy(bufs[s], out_hbm.at[b, pl.ds(i * K, K)],
                                wsems[s])

    total = BPW * NCHUNK
    chunks = [(wid * BPW + j // NCHUNK, j % NCHUNK) for j in range(total)]
    g = {}
    w = {}
    g[0] = copy_in(*chunks[0], 0)
    g[1] = copy_in(*chunks[1], 1)
    for j in range(total):
        s = j % 3
        if j + 2 < total:
            if j - 1 >= 0:
                w[j - 1].wait()
            g[j + 2] = copy_in(*chunks[j + 2], (j + 2) % 3)
        g[j].wait()
        w[j] = copy_out(*chunks[j], s)
    w[total - 3].wait()
    w[total - 2].wait()
    w[total - 1].wait()


@jax.jit
def _permute(x, perm):
    mesh = plsc.VectorSubcoreMesh(core_axis_name="c", subcore_axis_name="s")
    run = functools.partial(
        pl.kernel,
        mesh=mesh,
        compiler_params=pltpu.CompilerParams(use_tc_tiling_on_sc=False),
        out_type=jax.ShapeDtypeStruct((B, C, H * W), jnp.float32),
        scratch_types=[
            pltpu.VMEM((C,), jnp.int32),
            pltpu.VMEM((K, H * W), jnp.float32),
            pltpu.VMEM((K, H * W), jnp.float32),
            pltpu.VMEM((K, H * W), jnp.float32),
            pltpu.SemaphoreType.DMA,
            pltpu.SemaphoreType.DMA,
            pltpu.SemaphoreType.DMA,
            pltpu.SemaphoreType.DMA,
            pltpu.SemaphoreType.DMA,
            pltpu.SemaphoreType.DMA,
        ],
    )(_body)
    return run(x.reshape(B, C, H * W), perm)


def kernel(x, perm):
    y = _permute(x, perm).reshape(B, C, H, W)
    logdet = jnp.zeros((B,), dtype=x.dtype)
    return (y, logdet)
